# baseline (speedup 1.0000x reference)
import functools

import jax
import jax.numpy as jnp
from jax import lax
from jax.experimental import pallas as pl
from jax.experimental.pallas import tpu as pltpu
from jax.experimental.pallas import tpu_sc as plsc

N = 10000
K = 32
D = 128
LAT = 64

NC = 2
NS = 16
NW = NC * NS
NODES_PW = 320
NPAD = NW * NODES_PW
CHUNK = 4
RPC = CHUNK * K
NCH = NODES_PW // CHUNK
NBUF = 4
INV_K = 1.0 / K

@functools.cache
def _build_gather_mean():
    mesh = plsc.VectorSubcoreMesh(core_axis_name="c", subcore_axis_name="s")
    return functools.partial(
        pl.kernel,
        mesh=mesh,
        out_type=jax.ShapeDtypeStruct((NPAD, D), jnp.float32),
        scratch_types=[
            pltpu.VMEM((NODES_PW * K,), jnp.int32),
            pltpu.VMEM((NBUF, RPC, D), jnp.float32),
            pltpu.VMEM((NBUF, CHUNK, D), jnp.float32),
        ] + [pltpu.SemaphoreType.DMA] * (2 * NBUF),
    )(_gather_mean_body)


def _gather_mean(p, idx):
    return _build_gather_mean()(p, idx)


def _gather_mean_body(p_hbm, idx_hbm, out_hbm, idx_v, rows_v, stage_v, *sems):
    wid = lax.axis_index("s") * NC + lax.axis_index("c")
    node_base = wid * NODES_PW

    pltpu.sync_copy(idx_hbm.at[pl.ds(node_base * K, NODES_PW * K)], idx_v)

    gsems = sems[:NBUF]
    osems = sems[NBUF:]

    def start_gather(c, b):
        pltpu.async_copy(
            p_hbm.at[idx_v.at[pl.ds(c * RPC, RPC)]], rows_v.at[b], gsems[b])

    def wait_gather(b):
        pltpu.make_async_copy(
            p_hbm.at[pl.ds(0, RPC)], rows_v.at[b], gsems[b]).wait()

    def start_out(c, b):
        pltpu.async_copy(
            stage_v.at[b], out_hbm.at[pl.ds(node_base + c * CHUNK, CHUNK)], osems[b])

    def wait_out(b):
        pltpu.make_async_copy(
            stage_v.at[b], out_hbm.at[pl.ds(0, CHUNK)], osems[b]).wait()

    def accumulate(b):
        for j in range(CHUNK):
            base = j * K
            for cc in range(D // 16):
                col = pl.ds(cc * 16, 16)
                parts = [rows_v[b, base + u, col] for u in range(4)]
                for k in range(4, K, 4):
                    for u in range(4):
                        parts[u] = parts[u] + rows_v[b, base + k + u, col]
                acc = (parts[0] + parts[1]) + (parts[2] + parts[3])
                stage_v[b, j, col] = acc * INV_K

    for b in range(NBUF):
        start_gather(b, b)

    def body(i, carry):
        for b in range(NBUF):
            c = NBUF * i + b
            wait_gather(b)

            @pl.when(i > 0)
            def _():
                wait_out(b)

            accumulate(b)

            @pl.when(c + NBUF < NCH)
            def _():
                start_gather(c + NBUF, b)

            start_out(c, b)
        return carry

    lax.fori_loop(0, NCH // NBUF, body, 0)
    for b in range(NBUF):
        wait_out(b)


_BLK = 2000
_GRID = N // _BLK


def _row_spec(cols):
    return pl.BlockSpec((_BLK, cols), lambda i: (i, 0))


def _full_spec(r, c):
    return pl.BlockSpec((r, c), lambda i: (0, 0))


def _dot(a, b):
    return jnp.dot(a, b, preferred_element_type=jnp.float32)


def _tc_lift_body(x_ref, ci_ref, wl_ref, wci_ref, wn_ref, h_ref, p_ref):
    h = x_ref[...] * wl_ref[...] + _dot(ci_ref[...], wci_ref[...])
    h_ref[...] = h
    p_ref[...] = _dot(h, wn_ref[...])


def _tc_lift(xc, ci, wl, wci, wn):
    return pl.pallas_call(
        _tc_lift_body,
        grid=(_GRID,),
        in_specs=[_row_spec(1), _row_spec(2), _full_spec(1, D), _full_spec(2, D),
                  _full_spec(D, D)],
        out_specs=[_row_spec(D), _row_spec(D)],
        out_shape=[jax.ShapeDtypeStruct((N, D), jnp.float32)] * 2,
    )(xc, ci, wl, wci, wn)


def _tc_mp_body(h_ref, m_ref, ws_ref, wn_ref, h_out_ref, p_out_ref):
    hn = jax.nn.gelu(_dot(h_ref[...], ws_ref[...]) + m_ref[...])
    h_out_ref[...] = hn
    p_out_ref[...] = _dot(hn, wn_ref[...])


def _tc_mp(h, m, ws, wn):
    return pl.pallas_call(
        _tc_mp_body,
        grid=(_GRID,),
        in_specs=[_row_spec(D), _row_spec(D), _full_spec(D, D), _full_spec(D, D)],
        out_specs=[_row_spec(D), _row_spec(D)],
        out_shape=[jax.ShapeDtypeStruct((N, D), jnp.float32)] * 2,
    )(h, m, ws, wn)


def _tc_mid_body(h_ref, m_ref, ws_ref, wq_ref, wpost_ref, co_ref, wco_ref,
                 eps_ref, wn_ref, mom_ref, g_ref, p_ref):
    h2 = jax.nn.gelu(_dot(h_ref[...], ws_ref[...]) + m_ref[...])
    mom = _dot(h2, wq_ref[...])
    mu = mom[:, :LAT]
    logvar = jnp.clip(mom[:, LAT:], -30.0, 20.0)
    z = mu + jnp.exp(0.5 * logvar) * eps_ref[...]
    g = _dot(z, wpost_ref[...]) + _dot(co_ref[...], wco_ref[...])
    mom_ref[...] = jnp.concatenate([mu, logvar], axis=1)
    g_ref[...] = g
    p_ref[...] = _dot(g, wn_ref[...])


def _tc_mid(h, m, ws, wq, wpost, co, wco, eps, wn):
    return pl.pallas_call(
        _tc_mid_body,
        grid=(_GRID,),
        in_specs=[_row_spec(D), _row_spec(D), _full_spec(D, D),
                  _full_spec(D, 2 * LAT), _full_spec(LAT, D), _row_spec(2),
                  _full_spec(2, D), _row_spec(LAT), _full_spec(D, D)],
        out_specs=[_row_spec(2 * LAT), _row_spec(D), _row_spec(D)],
        out_shape=[jax.ShapeDtypeStruct((N, 2 * LAT), jnp.float32),
                   jax.ShapeDtypeStruct((N, D), jnp.float32),
                   jax.ShapeDtypeStruct((N, D), jnp.float32)],
    )(h, m, ws, wq, wpost, co, wco, eps, wn)


def _tc_out_body(g_ref, m_ref, ws_ref, wout_ref, dec_ref):
    g2 = jax.nn.gelu(_dot(g_ref[...], ws_ref[...]) + m_ref[...])
    dec_ref[...] = jnp.sum(g2 * wout_ref[...], axis=1, keepdims=True)


def _tc_out(g, m, ws, wout_row):
    return pl.pallas_call(
        _tc_out_body,
        grid=(_GRID,),
        in_specs=[_row_spec(D), _row_spec(D), _full_spec(D, D), _full_spec(1, D)],
        out_specs=[_row_spec(1)],
        out_shape=[jax.ShapeDtypeStruct((N, 1), jnp.float32)],
    )(g, m, ws, wout_row)[0]


def kernel(x, coords_input, coords_output, adjc, W_lift, W_coord_in, W_coord_out,
           W_es1, W_en1, W_es2, W_en2, W_q, W_post,
           W_ds1, W_dn1, W_ds2, W_dn2, W_out, eps):
    b = x.shape[0]
    xc = x.reshape(N, 1)
    idx = jnp.pad(adjc.reshape(-1), (0, NPAD * K - N * K))

    h0, p0 = _tc_lift(xc, coords_input, W_lift, W_coord_in, W_en1)
    m1 = _gather_mean(p0, idx)[:N]
    h1, p1 = _tc_mp(h0, m1, W_es1, W_en2)
    m2 = _gather_mean(p1, idx)[:N]
    mom, g0, p2 = _tc_mid(h1, m2, W_es2, W_q, W_post, coords_output,
                          W_coord_out, eps, W_dn1)
    m3 = _gather_mean(p2, idx)[:N]
    g1, p3 = _tc_mp(g0, m3, W_ds1, W_dn2)
    m4 = _gather_mean(p3, idx)[:N]
    dec = _tc_out(g1, m4, W_ds2, W_out.reshape(1, D))

    return dec.reshape(b, N, -1), mom[:, :LAT], mom[:, LAT:]


# speedup vs baseline: 3.9734x; 3.9734x over previous
import functools

import jax
import jax.numpy as jnp
from jax import lax
from jax.experimental import pallas as pl
from jax.experimental.pallas import tpu as pltpu
from jax.experimental.pallas import tpu_sc as plsc

N = 10000
K = 32
D = 128
LAT = 64

NC = 2
NS = 16
NW = NC * NS
RPW = D // NW
NBLK = 512
NPAD = 10240
NB = NPAD // NBLK
INV_K = 1.0 / K


@functools.cache
def _build_gather_mean():
    mesh = plsc.VectorSubcoreMesh(core_axis_name="c", subcore_axis_name="s")
    return functools.partial(
        pl.kernel,
        mesh=mesh,
        compiler_params=pltpu.CompilerParams(needs_layout_passes=False),
        out_type=jax.ShapeDtypeStruct((D, NPAD), jnp.float32),
        scratch_types=[
            pltpu.VMEM((RPW * NPAD,), jnp.float32),
            pltpu.VMEM((2, K, NBLK), jnp.int32),
            pltpu.VMEM((2, RPW, NBLK), jnp.float32),
            pltpu.SemaphoreType.DMA,
            pltpu.SemaphoreType.DMA,
            pltpu.SemaphoreType.DMA,
            pltpu.SemaphoreType.DMA,
        ],
    )(_gather_mean_body)


def _gather_mean(pt, adjt):
    return _build_gather_mean()(pt, adjt)


def _gather_mean_body(pt_hbm, adjt_hbm, out_hbm, tab_v, idx_v, stage_v, *sems):
    wid = lax.axis_index("s") * NC + lax.axis_index("c")
    row0 = wid * RPW

    for u in range(RPW):
        pltpu.sync_copy(pt_hbm.at[row0 + u], tab_v.at[pl.ds(u * NPAD, NPAD)])

    isems = sems[:2]
    osems = sems[2:]

    def start_idx(nb, b):
        pltpu.async_copy(
            adjt_hbm.at[pl.ds(0, K), pl.ds(nb * NBLK, NBLK)], idx_v.at[b], isems[b])

    def wait_idx(b):
        pltpu.make_async_copy(
            adjt_hbm.at[pl.ds(0, K), pl.ds(0, NBLK)], idx_v.at[b], isems[b]).wait()

    def start_out(nb, b):
        pltpu.async_copy(
            stage_v.at[b],
            out_hbm.at[pl.ds(row0, RPW), pl.ds(nb * NBLK, NBLK)], osems[b])

    def wait_out(b):
        pltpu.make_async_copy(
            stage_v.at[b], out_hbm.at[pl.ds(0, RPW), pl.ds(0, NBLK)], osems[b]).wait()

    def compute(b):
        def gbody(g, carry):
            lanes = pl.ds(g * 16, 16)
            accs = [jnp.zeros((16,), jnp.float32) for _ in range(RPW)]
            for k in range(K):
                ivec = idx_v[b, k, lanes]
                for u in range(RPW):
                    flat = ivec + (u * NPAD) if u else ivec
                    accs[u] = accs[u] + plsc.load_gather(tab_v, [flat])
            for u in range(RPW):
                stage_v[b, u, lanes] = accs[u] * INV_K
            return carry

        lax.fori_loop(0, NBLK // 16, gbody, 0)

    start_idx(0, 0)
    start_idx(1, 1)

    def body(i, carry):
        for b in range(2):
            nb = 2 * i + b
            wait_idx(b)

            @pl.when(i > 0)
            def _():
                wait_out(b)

            compute(b)

            @pl.when(nb + 2 < NB)
            def _():
                start_idx(nb + 2, b)

            start_out(nb, b)
        return carry

    lax.fori_loop(0, NB // 2, body, 0)
    wait_out(0)
    wait_out(1)


_BLK = 2048
_GRID = NPAD // _BLK


def _col_spec(rows):
    return pl.BlockSpec((rows, _BLK), lambda i: (0, i))


def _full_spec(r, c):
    return pl.BlockSpec((r, c), lambda i: (0, 0))


def _dot_t(w, xt):
    return lax.dot_general(w, xt, (((0,), (0,)), ((), ())),
                           preferred_element_type=jnp.float32)


def _tc_lift_body(xt_ref, cit_ref, wl_ref, wci_ref, wn_ref, ht_ref, pt_ref):
    ht = _dot_t(wl_ref[...], xt_ref[...]) + _dot_t(wci_ref[...], cit_ref[...])
    ht_ref[...] = ht
    pt_ref[...] = _dot_t(wn_ref[...], ht)


def _tc_lift(xt, cit, wl, wci, wn):
    return pl.pallas_call(
        _tc_lift_body,
        grid=(_GRID,),
        in_specs=[_col_spec(1), _col_spec(2), _full_spec(1, D), _full_spec(2, D),
                  _full_spec(D, D)],
        out_specs=[_col_spec(D), _col_spec(D)],
        out_shape=[jax.ShapeDtypeStruct((D, NPAD), jnp.float32)] * 2,
    )(xt, cit, wl, wci, wn)


def _tc_mp_body(ht_ref, mt_ref, ws_ref, wn_ref, ht_out_ref, pt_out_ref):
    hnt = jax.nn.gelu(_dot_t(ws_ref[...], ht_ref[...]) + mt_ref[...])
    ht_out_ref[...] = hnt
    pt_out_ref[...] = _dot_t(wn_ref[...], hnt)


def _tc_mp(ht, mt, ws, wn):
    return pl.pallas_call(
        _tc_mp_body,
        grid=(_GRID,),
        in_specs=[_col_spec(D), _col_spec(D), _full_spec(D, D), _full_spec(D, D)],
        out_specs=[_col_spec(D), _col_spec(D)],
        out_shape=[jax.ShapeDtypeStruct((D, NPAD), jnp.float32)] * 2,
    )(ht, mt, ws, wn)


def _tc_mid_body(ht_ref, mt_ref, ws_ref, wq_ref, wpost_ref, cot_ref, wco_ref,
                 epst_ref, wn_ref, momt_ref, gt_ref, pt_ref):
    h2t = jax.nn.gelu(_dot_t(ws_ref[...], ht_ref[...]) + mt_ref[...])
    momt = _dot_t(wq_ref[...], h2t)
    mut = momt[:LAT]
    logvart = jnp.clip(momt[LAT:], -30.0, 20.0)
    zt = mut + jnp.exp(0.5 * logvart) * epst_ref[...]
    gt = _dot_t(wpost_ref[...], zt) + _dot_t(wco_ref[...], cot_ref[...])
    momt_ref[...] = jnp.concatenate([mut, logvart], axis=0)
    gt_ref[...] = gt
    pt_ref[...] = _dot_t(wn_ref[...], gt)


def _tc_mid(ht, mt, ws, wq, wpost, cot, wco, epst, wn):
    return pl.pallas_call(
        _tc_mid_body,
        grid=(_GRID,),
        in_specs=[_col_spec(D), _col_spec(D), _full_spec(D, D),
                  _full_spec(D, 2 * LAT), _full_spec(LAT, D), _col_spec(2),
                  _full_spec(2, D), _col_spec(LAT), _full_spec(D, D)],
        out_specs=[_col_spec(2 * LAT), _col_spec(D), _col_spec(D)],
        out_shape=[jax.ShapeDtypeStruct((2 * LAT, NPAD), jnp.float32),
                   jax.ShapeDtypeStruct((D, NPAD), jnp.float32),
                   jax.ShapeDtypeStruct((D, NPAD), jnp.float32)],
    )(ht, mt, ws, wq, wpost, cot, wco, epst, wn)


def _tc_out_body(gt_ref, mt_ref, ws_ref, wout_ref, dect_ref):
    g2t = jax.nn.gelu(_dot_t(ws_ref[...], gt_ref[...]) + mt_ref[...])
    dect_ref[...] = jnp.sum(g2t * wout_ref[...], axis=0, keepdims=True)


def _tc_out(gt, mt, ws, wout):
    return pl.pallas_call(
        _tc_out_body,
        grid=(_GRID,),
        in_specs=[_col_spec(D), _col_spec(D), _full_spec(D, D), _full_spec(D, 1)],
        out_specs=[_col_spec(1)],
        out_shape=[jax.ShapeDtypeStruct((1, NPAD), jnp.float32)],
    )(gt, mt, ws, wout)[0]


def kernel(x, coords_input, coords_output, adjc, W_lift, W_coord_in, W_coord_out,
           W_es1, W_en1, W_es2, W_en2, W_q, W_post,
           W_ds1, W_dn1, W_ds2, W_dn2, W_out, eps):
    b = x.shape[0]
    pad = ((0, 0), (0, NPAD - N))
    xt = jnp.pad(x.reshape(1, N), pad)
    cit = jnp.pad(coords_input.T, pad)
    cot = jnp.pad(coords_output.T, pad)
    epst = jnp.pad(eps.T, pad)
    adjt = jnp.pad(adjc.T, pad)

    h0t, p0t = _tc_lift(xt, cit, W_lift, W_coord_in, W_en1)
    m1t = _gather_mean(p0t, adjt)
    h1t, p1t = _tc_mp(h0t, m1t, W_es1, W_en2)
    m2t = _gather_mean(p1t, adjt)
    momt, g0t, p2t = _tc_mid(h1t, m2t, W_es2, W_q, W_post, cot, W_coord_out,
                             epst, W_dn1)
    m3t = _gather_mean(p2t, adjt)
    g1t, p3t = _tc_mp(g0t, m3t, W_ds1, W_dn2)
    m4t = _gather_mean(p3t, adjt)
    dect = _tc_out(g1t, m4t, W_ds2, W_out)

    return (dect[:, :N].reshape(b, N, 1), momt[:LAT, :N].T,
            momt[LAT:, :N].T)


# speedup vs baseline: 5.3906x; 1.3567x over previous
import functools

import jax
import jax.numpy as jnp
from jax import lax
from jax.experimental import pallas as pl
from jax.experimental.pallas import tpu as pltpu
from jax.experimental.pallas import tpu_sc as plsc

N = 10000
K = 32
D = 128
LAT = 64

NC = 2
NS = 16
NW = NC * NS
RPW = D // NW
NBLK = 512
NPAD = 10240
NB = NPAD // NBLK
INV_K = 1.0 / K


@functools.cache
def _build_gather_mean():
    mesh = plsc.VectorSubcoreMesh(core_axis_name="c", subcore_axis_name="s")
    return functools.partial(
        pl.kernel,
        mesh=mesh,
        compiler_params=pltpu.CompilerParams(needs_layout_passes=False),
        out_type=jax.ShapeDtypeStruct((D, NPAD), jnp.float32),
        scratch_types=[
            pltpu.VMEM((RPW // 2 * NPAD,), jnp.float32),
            pltpu.VMEM((2, K, NBLK), jnp.int32),
            pltpu.VMEM((2, RPW, NBLK), jnp.float32),
            pltpu.SemaphoreType.DMA,
            pltpu.SemaphoreType.DMA,
            pltpu.SemaphoreType.DMA,
            pltpu.SemaphoreType.DMA,
        ],
    )(_gather_mean_body)


def _gather_mean(pt, adjt):
    return _build_gather_mean()(pt, adjt)


def _gather_mean_body(pt_hbm, adjt_hbm, out_hbm, tab_v, idx_v, stage_v, *sems):
    wid = lax.axis_index("s") * NC + lax.axis_index("c")
    row0 = wid * RPW

    for v in range(RPW // 2):
        pltpu.sync_copy(pt_hbm.at[wid * (RPW // 2) + v],
                        tab_v.at[pl.ds(v * NPAD, NPAD)])

    isems = sems[:2]
    osems = sems[2:]

    def start_idx(nb, b):
        pltpu.async_copy(
            adjt_hbm.at[pl.ds(0, K), pl.ds(nb * NBLK, NBLK)], idx_v.at[b], isems[b])

    def wait_idx(b):
        pltpu.make_async_copy(
            adjt_hbm.at[pl.ds(0, K), pl.ds(0, NBLK)], idx_v.at[b], isems[b]).wait()

    def start_out(nb, b):
        pltpu.async_copy(
            stage_v.at[b],
            out_hbm.at[pl.ds(row0, RPW), pl.ds(nb * NBLK, NBLK)], osems[b])

    def wait_out(b):
        pltpu.make_async_copy(
            stage_v.at[b], out_hbm.at[pl.ds(0, RPW), pl.ds(0, NBLK)], osems[b]).wait()

    def compute(b):
        def gbody(g, carry):
            lanes = pl.ds(g * 16, 16)
            accs = [jnp.zeros((16,), jnp.float32) for _ in range(RPW)]
            for k in range(K):
                ivec = idx_v[b, k, lanes]
                for v in range(RPW // 2):
                    flat = ivec + (v * NPAD) if v else ivec
                    packed = plsc.load_gather(tab_v, [flat])
                    even, odd = plsc.unpack(
                        plsc.bitcast(packed, jnp.bfloat16),
                        format=plsc.PackFormat.INTERLEAVED)
                    accs[2 * v] = accs[2 * v] + even
                    accs[2 * v + 1] = accs[2 * v + 1] + odd
            for u in range(RPW):
                stage_v[b, u, lanes] = accs[u] * INV_K
            return carry

        lax.fori_loop(0, NBLK // 16, gbody, 0)

    start_idx(0, 0)
    start_idx(1, 1)

    def body(i, carry):
        for b in range(2):
            nb = 2 * i + b
            wait_idx(b)

            @pl.when(i > 0)
            def _():
                wait_out(b)

            compute(b)

            @pl.when(nb + 2 < NB)
            def _():
                start_idx(nb + 2, b)

            start_out(nb, b)
        return carry

    lax.fori_loop(0, NB // 2, body, 0)
    wait_out(0)
    wait_out(1)


_BLK = 2048
_GRID = NPAD // _BLK


def _col_spec(rows):
    return pl.BlockSpec((rows, _BLK), lambda i: (0, i))


def _full_spec(r, c):
    return pl.BlockSpec((r, c), lambda i: (0, 0))


def _dot_t(w, xt):
    return lax.dot_general(w, xt, (((0,), (0,)), ((), ())),
                           preferred_element_type=jnp.float32)


def _pack_pairs(lo, hi):
    lo_u = lax.bitcast_convert_type(
        lax.convert_element_type(lo, jnp.bfloat16), jnp.uint16).astype(jnp.uint32)
    hi_u = lax.bitcast_convert_type(
        lax.convert_element_type(hi, jnp.bfloat16), jnp.uint16).astype(jnp.uint32)
    return lax.bitcast_convert_type((hi_u << 16) | lo_u, jnp.float32)


def _proj_packed(wn_e, wn_o, ht):
    return _pack_pairs(_dot_t(wn_e, ht), _dot_t(wn_o, ht))


def _tc_lift_body(xt_ref, cit_ref, wl_ref, wci_ref, wne_ref, wno_ref,
                  ht_ref, pt_ref):
    ht = _dot_t(wl_ref[...], xt_ref[...]) + _dot_t(wci_ref[...], cit_ref[...])
    ht_ref[...] = ht
    pt_ref[...] = _proj_packed(wne_ref[...], wno_ref[...], ht)


def _tc_lift(xt, cit, wl, wci, wne, wno):
    return pl.pallas_call(
        _tc_lift_body,
        grid=(_GRID,),
        in_specs=[_col_spec(1), _col_spec(2), _full_spec(1, D), _full_spec(2, D),
                  _full_spec(D, D // 2), _full_spec(D, D // 2)],
        out_specs=[_col_spec(D), _col_spec(D // 2)],
        out_shape=[jax.ShapeDtypeStruct((D, NPAD), jnp.float32),
                   jax.ShapeDtypeStruct((D // 2, NPAD), jnp.float32)],
    )(xt, cit, wl, wci, wne, wno)


def _tc_mp_body(ht_ref, mt_ref, ws_ref, wne_ref, wno_ref, ht_out_ref,
                pt_out_ref):
    hnt = jax.nn.gelu(_dot_t(ws_ref[...], ht_ref[...]) + mt_ref[...])
    ht_out_ref[...] = hnt
    pt_out_ref[...] = _proj_packed(wne_ref[...], wno_ref[...], hnt)


def _tc_mp(ht, mt, ws, wne, wno):
    return pl.pallas_call(
        _tc_mp_body,
        grid=(_GRID,),
        in_specs=[_col_spec(D), _col_spec(D), _full_spec(D, D),
                  _full_spec(D, D // 2), _full_spec(D, D // 2)],
        out_specs=[_col_spec(D), _col_spec(D // 2)],
        out_shape=[jax.ShapeDtypeStruct((D, NPAD), jnp.float32),
                   jax.ShapeDtypeStruct((D // 2, NPAD), jnp.float32)],
    )(ht, mt, ws, wne, wno)


def _tc_mid_body(ht_ref, mt_ref, ws_ref, wq_ref, wpost_ref, cot_ref, wco_ref,
                 epst_ref, wne_ref, wno_ref, momt_ref, gt_ref, pt_ref):
    h2t = jax.nn.gelu(_dot_t(ws_ref[...], ht_ref[...]) + mt_ref[...])
    momt = _dot_t(wq_ref[...], h2t)
    mut = momt[:LAT]
    logvart = jnp.clip(momt[LAT:], -30.0, 20.0)
    zt = mut + jnp.exp(0.5 * logvart) * epst_ref[...]
    gt = _dot_t(wpost_ref[...], zt) + _dot_t(wco_ref[...], cot_ref[...])
    momt_ref[...] = jnp.concatenate([mut, logvart], axis=0)
    gt_ref[...] = gt
    pt_ref[...] = _proj_packed(wne_ref[...], wno_ref[...], gt)


def _tc_mid(ht, mt, ws, wq, wpost, cot, wco, epst, wne, wno):
    return pl.pallas_call(
        _tc_mid_body,
        grid=(_GRID,),
        in_specs=[_col_spec(D), _col_spec(D), _full_spec(D, D),
                  _full_spec(D, 2 * LAT), _full_spec(LAT, D), _col_spec(2),
                  _full_spec(2, D), _col_spec(LAT), _full_spec(D, D // 2),
                  _full_spec(D, D // 2)],
        out_specs=[_col_spec(2 * LAT), _col_spec(D), _col_spec(D // 2)],
        out_shape=[jax.ShapeDtypeStruct((2 * LAT, NPAD), jnp.float32),
                   jax.ShapeDtypeStruct((D, NPAD), jnp.float32),
                   jax.ShapeDtypeStruct((D // 2, NPAD), jnp.float32)],
    )(ht, mt, ws, wq, wpost, cot, wco, epst, wne, wno)


def _tc_out_body(gt_ref, mt_ref, ws_ref, wout_ref, dect_ref):
    g2t = jax.nn.gelu(_dot_t(ws_ref[...], gt_ref[...]) + mt_ref[...])
    dect_ref[...] = jnp.sum(g2t * wout_ref[...], axis=0, keepdims=True)


def _tc_out(gt, mt, ws, wout):
    return pl.pallas_call(
        _tc_out_body,
        grid=(_GRID,),
        in_specs=[_col_spec(D), _col_spec(D), _full_spec(D, D), _full_spec(D, 1)],
        out_specs=[_col_spec(1)],
        out_shape=[jax.ShapeDtypeStruct((1, NPAD), jnp.float32)],
    )(gt, mt, ws, wout)[0]


def kernel(x, coords_input, coords_output, adjc, W_lift, W_coord_in, W_coord_out,
           W_es1, W_en1, W_es2, W_en2, W_q, W_post,
           W_ds1, W_dn1, W_ds2, W_dn2, W_out, eps):
    b = x.shape[0]
    pad = ((0, 0), (0, NPAD - N))
    xt = jnp.pad(x.reshape(1, N), pad)
    cit = jnp.pad(coords_input.T, pad)
    cot = jnp.pad(coords_output.T, pad)
    epst = jnp.pad(eps.T, pad)
    adjt = jnp.pad(adjc.T, pad)

    wn_eo = [(w[:, 0::2], w[:, 1::2]) for w in (W_en1, W_en2, W_dn1, W_dn2)]

    h0t, p0t = _tc_lift(xt, cit, W_lift, W_coord_in, *wn_eo[0])
    m1t = _gather_mean(p0t, adjt)
    h1t, p1t = _tc_mp(h0t, m1t, W_es1, *wn_eo[1])
    m2t = _gather_mean(p1t, adjt)
    momt, g0t, p2t = _tc_mid(h1t, m2t, W_es2, W_q, W_post, cot, W_coord_out,
                             epst, *wn_eo[2])
    m3t = _gather_mean(p2t, adjt)
    g1t, p3t = _tc_mp(g0t, m3t, W_ds1, *wn_eo[3])
    m4t = _gather_mean(p3t, adjt)
    dect = _tc_out(g1t, m4t, W_ds2, W_out)

    return (dect[:, :N].reshape(b, N, 1), momt[:LAT, :N].T,
            momt[LAT:, :N].T)


# speedup vs baseline: 5.4284x; 1.0070x over previous
import functools

import jax
import jax.numpy as jnp
from jax import lax
from jax.experimental import pallas as pl
from jax.experimental.pallas import tpu as pltpu
from jax.experimental.pallas import tpu_sc as plsc

N = 10000
K = 32
D = 128
LAT = 64

NC = 2
NS = 16
NW = NC * NS
RPW = D // NW
NBLK = 512
NPAD = 10240
NB = NPAD // NBLK
INV_K = 1.0 / K


@functools.cache
def _build_gather_mean():
    mesh = plsc.VectorSubcoreMesh(core_axis_name="c", subcore_axis_name="s")
    return functools.partial(
        pl.kernel,
        mesh=mesh,
        compiler_params=pltpu.CompilerParams(needs_layout_passes=False),
        out_type=jax.ShapeDtypeStruct((D, NPAD), jnp.float32),
        scratch_types=[
            pltpu.VMEM((RPW // 2 * NPAD,), jnp.float32),
            pltpu.VMEM((2, K // 2, NBLK), jnp.int32),
            pltpu.VMEM((2, RPW, NBLK), jnp.float32),
            pltpu.SemaphoreType.DMA,
            pltpu.SemaphoreType.DMA,
            pltpu.SemaphoreType.DMA,
            pltpu.SemaphoreType.DMA,
        ],
    )(_gather_mean_body)


def _gather_mean(pt, adjt):
    return _build_gather_mean()(pt, adjt)


def _gather_mean_body(pt_hbm, adjt_hbm, out_hbm, tab_v, idx_v, stage_v, *sems):
    wid = lax.axis_index("s") * NC + lax.axis_index("c")
    row0 = wid * RPW

    for v in range(RPW // 2):
        pltpu.sync_copy(pt_hbm.at[wid * (RPW // 2) + v],
                        tab_v.at[pl.ds(v * NPAD, NPAD)])

    isems = sems[:2]
    osems = sems[2:]

    def start_idx(nb, b):
        pltpu.async_copy(
            adjt_hbm.at[pl.ds(0, K // 2), pl.ds(nb * NBLK, NBLK)], idx_v.at[b],
            isems[b])

    def wait_idx(b):
        pltpu.make_async_copy(
            adjt_hbm.at[pl.ds(0, K // 2), pl.ds(0, NBLK)], idx_v.at[b],
            isems[b]).wait()

    def start_out(nb, b):
        pltpu.async_copy(
            stage_v.at[b],
            out_hbm.at[pl.ds(row0, RPW), pl.ds(nb * NBLK, NBLK)], osems[b])

    def wait_out(b):
        pltpu.make_async_copy(
            stage_v.at[b], out_hbm.at[pl.ds(0, RPW), pl.ds(0, NBLK)], osems[b]).wait()

    def compute(b):
        def one_group(g16):
            lanes = pl.ds(g16 * 16, 16)
            accs = [jnp.zeros((16,), jnp.float32) for _ in range(RPW)]
            for j in range(K // 2):
                ipair = idx_v[b, j, lanes]
                ivecs = plsc.unpack(
                    plsc.bitcast(ipair, jnp.int16),
                    format=plsc.PackFormat.INTERLEAVED)
                for ivec in ivecs:
                    for v in range(RPW // 2):
                        flat = ivec + (v * NPAD) if v else ivec
                        packed = plsc.load_gather(tab_v, [flat])
                        even, odd = plsc.unpack(
                            plsc.bitcast(packed, jnp.bfloat16),
                            format=plsc.PackFormat.INTERLEAVED)
                        accs[2 * v] = accs[2 * v] + even
                        accs[2 * v + 1] = accs[2 * v + 1] + odd
            for u in range(RPW):
                stage_v[b, u, lanes] = accs[u] * INV_K

        def gbody(g, carry):
            one_group(2 * g)
            one_group(2 * g + 1)
            return carry

        lax.fori_loop(0, NBLK // 32, gbody, 0)

    start_idx(0, 0)
    start_idx(1, 1)

    def body(i, carry):
        for b in range(2):
            nb = 2 * i + b
            wait_idx(b)

            @pl.when(i > 0)
            def _():
                wait_out(b)

            compute(b)

            @pl.when(nb + 2 < NB)
            def _():
                start_idx(nb + 2, b)

            start_out(nb, b)
        return carry

    lax.fori_loop(0, NB // 2, body, 0)
    wait_out(0)
    wait_out(1)


_BLK = 2048
_GRID = NPAD // _BLK


def _col_spec(rows):
    return pl.BlockSpec((rows, _BLK), lambda i: (0, i))


def _full_spec(r, c):
    return pl.BlockSpec((r, c), lambda i: (0, 0))


def _dot_t(w, xt):
    return lax.dot_general(w, xt, (((0,), (0,)), ((), ())),
                           preferred_element_type=jnp.float32)


def _pack_pairs(lo, hi):
    lo_u = lax.bitcast_convert_type(
        lax.convert_element_type(lo, jnp.bfloat16), jnp.uint16).astype(jnp.uint32)
    hi_u = lax.bitcast_convert_type(
        lax.convert_element_type(hi, jnp.bfloat16), jnp.uint16).astype(jnp.uint32)
    return lax.bitcast_convert_type((hi_u << 16) | lo_u, jnp.float32)


def _proj_packed(wn_e, wn_o, ht):
    return _pack_pairs(_dot_t(wn_e, ht), _dot_t(wn_o, ht))


def _tc_lift_body(xt_ref, cit_ref, wl_ref, wci_ref, wne_ref, wno_ref,
                  ht_ref, pt_ref):
    ht = _dot_t(wl_ref[...], xt_ref[...]) + _dot_t(wci_ref[...], cit_ref[...])
    ht_ref[...] = ht
    pt_ref[...] = _proj_packed(wne_ref[...], wno_ref[...], ht)


def _tc_lift(xt, cit, wl, wci, wne, wno):
    return pl.pallas_call(
        _tc_lift_body,
        grid=(_GRID,),
        in_specs=[_col_spec(1), _col_spec(2), _full_spec(1, D), _full_spec(2, D),
                  _full_spec(D, D // 2), _full_spec(D, D // 2)],
        out_specs=[_col_spec(D), _col_spec(D // 2)],
        out_shape=[jax.ShapeDtypeStruct((D, NPAD), jnp.float32),
                   jax.ShapeDtypeStruct((D // 2, NPAD), jnp.float32)],
    )(xt, cit, wl, wci, wne, wno)


def _tc_mp_body(ht_ref, mt_ref, ws_ref, wne_ref, wno_ref, ht_out_ref,
                pt_out_ref):
    hnt = jax.nn.gelu(_dot_t(ws_ref[...], ht_ref[...]) + mt_ref[...])
    ht_out_ref[...] = hnt
    pt_out_ref[...] = _proj_packed(wne_ref[...], wno_ref[...], hnt)


def _tc_mp(ht, mt, ws, wne, wno):
    return pl.pallas_call(
        _tc_mp_body,
        grid=(_GRID,),
        in_specs=[_col_spec(D), _col_spec(D), _full_spec(D, D),
                  _full_spec(D, D // 2), _full_spec(D, D // 2)],
        out_specs=[_col_spec(D), _col_spec(D // 2)],
        out_shape=[jax.ShapeDtypeStruct((D, NPAD), jnp.float32),
                   jax.ShapeDtypeStruct((D // 2, NPAD), jnp.float32)],
    )(ht, mt, ws, wne, wno)


def _tc_mid_body(ht_ref, mt_ref, ws_ref, wq_ref, wpost_ref, cot_ref, wco_ref,
                 epst_ref, wne_ref, wno_ref, momt_ref, gt_ref, pt_ref):
    h2t = jax.nn.gelu(_dot_t(ws_ref[...], ht_ref[...]) + mt_ref[...])
    momt = _dot_t(wq_ref[...], h2t)
    mut = momt[:LAT]
    logvart = jnp.clip(momt[LAT:], -30.0, 20.0)
    zt = mut + jnp.exp(0.5 * logvart) * epst_ref[...]
    gt = _dot_t(wpost_ref[...], zt) + _dot_t(wco_ref[...], cot_ref[...])
    momt_ref[...] = jnp.concatenate([mut, logvart], axis=0)
    gt_ref[...] = gt
    pt_ref[...] = _proj_packed(wne_ref[...], wno_ref[...], gt)


def _tc_mid(ht, mt, ws, wq, wpost, cot, wco, epst, wne, wno):
    return pl.pallas_call(
        _tc_mid_body,
        grid=(_GRID,),
        in_specs=[_col_spec(D), _col_spec(D), _full_spec(D, D),
                  _full_spec(D, 2 * LAT), _full_spec(LAT, D), _col_spec(2),
                  _full_spec(2, D), _col_spec(LAT), _full_spec(D, D // 2),
                  _full_spec(D, D // 2)],
        out_specs=[_col_spec(2 * LAT), _col_spec(D), _col_spec(D // 2)],
        out_shape=[jax.ShapeDtypeStruct((2 * LAT, NPAD), jnp.float32),
                   jax.ShapeDtypeStruct((D, NPAD), jnp.float32),
                   jax.ShapeDtypeStruct((D // 2, NPAD), jnp.float32)],
    )(ht, mt, ws, wq, wpost, cot, wco, epst, wne, wno)


def _tc_out_body(gt_ref, mt_ref, ws_ref, wout_ref, dect_ref):
    g2t = jax.nn.gelu(_dot_t(ws_ref[...], gt_ref[...]) + mt_ref[...])
    dect_ref[...] = jnp.sum(g2t * wout_ref[...], axis=0, keepdims=True)


def _tc_out(gt, mt, ws, wout):
    return pl.pallas_call(
        _tc_out_body,
        grid=(_GRID,),
        in_specs=[_col_spec(D), _col_spec(D), _full_spec(D, D), _full_spec(D, 1)],
        out_specs=[_col_spec(1)],
        out_shape=[jax.ShapeDtypeStruct((1, NPAD), jnp.float32)],
    )(gt, mt, ws, wout)[0]


def kernel(x, coords_input, coords_output, adjc, W_lift, W_coord_in, W_coord_out,
           W_es1, W_en1, W_es2, W_en2, W_q, W_post,
           W_ds1, W_dn1, W_ds2, W_dn2, W_out, eps):
    b = x.shape[0]
    pad = ((0, 0), (0, NPAD - N))
    xt = jnp.pad(x.reshape(1, N), pad)
    cit = jnp.pad(coords_input.T, pad)
    cot = jnp.pad(coords_output.T, pad)
    epst = jnp.pad(eps.T, pad)
    adjt_full = jnp.pad(adjc.T, pad)
    adjt = (adjt_full[1::2] << 16) | adjt_full[0::2]

    wn_eo = [(w[:, 0::2], w[:, 1::2]) for w in (W_en1, W_en2, W_dn1, W_dn2)]

    h0t, p0t = _tc_lift(xt, cit, W_lift, W_coord_in, *wn_eo[0])
    m1t = _gather_mean(p0t, adjt)
    h1t, p1t = _tc_mp(h0t, m1t, W_es1, *wn_eo[1])
    m2t = _gather_mean(p1t, adjt)
    momt, g0t, p2t = _tc_mid(h1t, m2t, W_es2, W_q, W_post, cot, W_coord_out,
                             epst, *wn_eo[2])
    m3t = _gather_mean(p2t, adjt)
    g1t, p3t = _tc_mp(g0t, m3t, W_ds1, *wn_eo[3])
    m4t = _gather_mean(p3t, adjt)
    dect = _tc_out(g1t, m4t, W_ds2, W_out)

    return (dect[:, :N].reshape(b, N, 1), momt[:LAT, :N].T,
            momt[LAT:, :N].T)
